# baseline (speedup 1.0000x reference)
import jax
import jax.numpy as jnp
from jax.experimental import pallas as pl
from jax.experimental.pallas import tpu as pltpu

EPS_LN = 1e-12
BLOCK_S = 2048


def _ln_body(we_ref, pe_ref, gamma_ref, beta_ref, out_ref):
    h = we_ref[0] + pe_ref[...]
    d = h.shape[-1]
    s1 = jnp.sum(h, axis=-1, keepdims=True)
    s2 = jnp.sum(h * h, axis=-1, keepdims=True)
    mean = s1 * (1.0 / d)
    var = s2 * (1.0 / d) - mean * mean
    inv = jax.lax.rsqrt(var + EPS_LN)
    out_ref[0] = (h - mean) * (inv * gamma_ref[...]) + beta_ref[...]


def kernel(word_embeddings, pe_table, ln_gamma, ln_beta):
    B, S, D = word_embeddings.shape
    n_s = S // BLOCK_S
    gamma2 = ln_gamma.reshape(1, D)
    beta2 = ln_beta.reshape(1, D)
    return pl.pallas_call(
        _ln_body,
        grid=(n_s, B),
        in_specs=[
            pl.BlockSpec((1, BLOCK_S, D), lambda s, b: (b, s, 0)),
            pl.BlockSpec((BLOCK_S, D), lambda s, b: (s, 0)),
            pl.BlockSpec((1, D), lambda s, b: (0, 0)),
            pl.BlockSpec((1, D), lambda s, b: (0, 0)),
        ],
        out_specs=pl.BlockSpec((1, BLOCK_S, D), lambda s, b: (b, s, 0)),
        out_shape=jax.ShapeDtypeStruct((B, S, D), jnp.float32),
        compiler_params=pltpu.CompilerParams(
            dimension_semantics=("parallel", "parallel"),
        ),
    )(word_embeddings, pe_table, gamma2, beta2)
